# Initial kernel scaffold; baseline (speedup 1.0000x reference)
import jax
import jax.numpy as jnp
from jax.experimental import pallas as pl


def kernel(xyzs, feats, W1, b1, g1, be1, W2, b2, g2, be2, W3, b3, g3, be3):
    raise NotImplementedError("write your pallas kernel here")


# speedup vs baseline: 12.2336x; 12.2336x over previous
import functools

import jax
import jax.numpy as jnp
import numpy as np
from jax import lax
from jax.experimental import pallas as pl
from jax.experimental.pallas import tpu as pltpu
from jax.experimental.pallas import tpu_sc as plsc

B = 8
N = 4096
S = 1024
K = 32
RTOT = B * S * K
R2 = np.float32(0.15) * np.float32(0.15)
EPSF = np.float32(1e-5)
RB = 4096
G = RTOT // RB
SB = 128
DPAD = 32


def _fps_body(xs_ref, ys_ref, zs_ref, idx_ref, mind_ref):
    X = xs_ref[...]
    Y = ys_ref[...]
    Z = zs_ref[...]
    lane = lax.broadcasted_iota(jnp.int32, (B, N), 1)
    mind_ref[...] = jnp.full((B, N), jnp.inf, dtype=jnp.float32)

    def body(i, carry):
        lx, ly, lz, acc = carry
        dx = X - lx
        dy = Y - ly
        dz = Z - lz
        d2 = dx * dx + dy * dy + dz * dz
        mind = jnp.minimum(mind_ref[...], d2)
        mind_ref[...] = mind
        m = jnp.max(mind, axis=1, keepdims=True)
        nxt = jnp.min(jnp.where(mind == m, lane, N), axis=1, keepdims=True)
        oh = lane == nxt
        lx = jnp.sum(jnp.where(oh, X, 0.0), axis=1, keepdims=True)
        ly = jnp.sum(jnp.where(oh, Y, 0.0), axis=1, keepdims=True)
        lz = jnp.sum(jnp.where(oh, Z, 0.0), axis=1, keepdims=True)
        si = lax.broadcasted_iota(jnp.int32, (B, S), 1)
        acc = jnp.where(si == i, nxt, acc)
        return lx, ly, lz, acc

    lx0 = X[:, 0:1]
    ly0 = Y[:, 0:1]
    lz0 = Z[:, 0:1]
    acc0 = jnp.zeros((B, S), dtype=jnp.int32)
    _, _, _, acc = lax.fori_loop(1, S, body, (lx0, ly0, lz0, acc0))
    idx_ref[...] = acc


def _fps(xs, ys, zs):
    return pl.pallas_call(
        _fps_body,
        out_shape=jax.ShapeDtypeStruct((B, S), jnp.int32),
        scratch_shapes=[pltpu.VMEM((B, N), jnp.float32)],
    )(xs, ys, zs)


def _bq_body(cx_ref, cy_ref, cz_ref, xs_ref, ys_ref, zs_ref, idx_ref):
    cx = cx_ref[0]
    cy = cy_ref[0]
    cz = cz_ref[0]
    X = xs_ref[0]
    Y = ys_ref[0]
    Z = zs_ref[0]
    dx = cx - X
    dy = cy - Y
    dz = cz - Z
    d2 = dx * dx + dy * dy + dz * dz
    within = d2 < R2
    lane = lax.broadcasted_iota(jnp.int32, (SB, N), 1)
    vals = jnp.where(within, lane, N)
    for k in range(K):
        m = jnp.min(vals, axis=1, keepdims=True)
        idx_ref[0, :, k : k + 1] = jnp.where(m < N, m, -1)
        vals = jnp.where(vals == m, N, vals)


def _ball_query_run(cx, cy, cz, xs, ys, zs):
    spec_c = pl.BlockSpec((1, SB, 1), lambda b, s: (b, s, 0))
    spec_x = pl.BlockSpec((1, 1, N), lambda b, s: (b, 0, 0))
    return pl.pallas_call(
        _bq_body,
        grid=(B, S // SB),
        in_specs=[spec_c, spec_c, spec_c, spec_x, spec_x, spec_x],
        out_specs=pl.BlockSpec((1, SB, K), lambda b, s: (b, s, 0)),
        out_shape=jax.ShapeDtypeStruct((B, S, K), jnp.int32),
    )(cx, cy, cz, xs[:, None, :], ys[:, None, :], zs[:, None, :])


def _sc_gather(table, flat_idx):
    info = plsc.get_sparse_core_info()
    NC, NS = info.num_cores, info.num_subcores
    NW = NC * NS
    per_w = RTOT // NW
    CH = 128
    OUT_CH = 1024
    n_outer = per_w // OUT_CH
    n_inner = OUT_CH // CH
    idx2d = flat_idx.reshape(RTOT // CH, CH)
    mesh = plsc.VectorSubcoreMesh(core_axis_name="c", subcore_axis_name="s")

    @functools.partial(
        pl.kernel,
        out_type=jax.ShapeDtypeStruct((RTOT, DPAD), jnp.float32),
        mesh=mesh,
        scratch_types=[
            pltpu.VMEM((n_inner, CH), jnp.int32),
            pltpu.VMEM((OUT_CH, DPAD), jnp.float32),
            pltpu.SemaphoreType.DMA,
        ],
        compiler_params=pltpu.CompilerParams(use_tc_tiling_on_sc=False),
    )
    def k(table_hbm, idx_hbm, out_hbm, idx_v, rows_v, sem):
        wid = lax.axis_index("s") * NC + lax.axis_index("c")
        row0 = wid * (per_w // CH)

        def outer(g, _):
            pltpu.sync_copy(idx_hbm.at[pl.ds(row0 + g * n_inner, n_inner)], idx_v)
            copies = []
            for j in range(n_inner):
                copies.append(pltpu.async_copy(
                    table_hbm.at[idx_v.at[j]],
                    rows_v.at[pl.ds(j * CH, CH)],
                    sem,
                ))
            for c in copies:
                c.wait()
            pltpu.sync_copy(
                rows_v, out_hbm.at[pl.ds(wid * per_w + g * OUT_CH, OUT_CH)])
            return 0

        lax.fori_loop(0, n_outer, outer, 0)

    return k(table, idx2d)


def _m1_body(g_ref, c_ref, m_ref, w_ref, wx_ref, b_ref, a_ref, ps_ref, pq_ref):
    gv = g_ref[...] * m_ref[...]
    cm = c_ref[...] * m_ref[...]
    a = (jnp.dot(gv, w_ref[...], preferred_element_type=jnp.float32)
         - jnp.dot(cm, wx_ref[...], preferred_element_type=jnp.float32)
         + b_ref[...])
    a_ref[...] = a
    ps_ref[0] = jnp.sum(a, axis=0, keepdims=True)
    pq_ref[0] = jnp.sum(a * a, axis=0, keepdims=True)


def _mlp1(gathered, crep, mask, w1p, wx, b1r):
    return pl.pallas_call(
        _m1_body,
        grid=(G,),
        in_specs=[
            pl.BlockSpec((RB, DPAD), lambda i: (i, 0)),
            pl.BlockSpec((RB, 8), lambda i: (i, 0)),
            pl.BlockSpec((RB, 1), lambda i: (i, 0)),
            pl.BlockSpec((DPAD, 32), lambda i: (0, 0)),
            pl.BlockSpec((8, 32), lambda i: (0, 0)),
            pl.BlockSpec((1, 32), lambda i: (0, 0)),
        ],
        out_specs=[
            pl.BlockSpec((RB, 32), lambda i: (i, 0)),
            pl.BlockSpec((1, 1, 32), lambda i: (i, 0, 0)),
            pl.BlockSpec((1, 1, 32), lambda i: (i, 0, 0)),
        ],
        out_shape=[
            jax.ShapeDtypeStruct((RTOT, 32), jnp.float32),
            jax.ShapeDtypeStruct((G, 1, 32), jnp.float32),
            jax.ShapeDtypeStruct((G, 1, 32), jnp.float32),
        ],
    )(gathered, crep, mask, w1p, wx, b1r)


def _mid_body(a_ref, ps_ref, pq_ref, g_ref, be_ref, w_ref, b_ref,
              o_ref, ops_ref, opq_ref):
    cnt = np.float32(RTOT)
    mean = jnp.sum(ps_ref[...], axis=0) / cnt
    ex2 = jnp.sum(pq_ref[...], axis=0) / cnt
    var = ex2 - mean * mean
    rstd = lax.rsqrt(var + EPSF)
    scale = g_ref[...] * rstd
    shift = be_ref[...] - mean * scale
    h = jnp.maximum(a_ref[...] * scale + shift, 0.0)
    o = jnp.dot(h, w_ref[...], preferred_element_type=jnp.float32) + b_ref[...]
    o_ref[...] = o
    ops_ref[0] = jnp.sum(o, axis=0, keepdims=True)
    opq_ref[0] = jnp.sum(o * o, axis=0, keepdims=True)


def _mlp_mid(a, ps, pq, gr, ber, wp, br, cout):
    cin = a.shape[1]
    return pl.pallas_call(
        _mid_body,
        grid=(G,),
        in_specs=[
            pl.BlockSpec((RB, cin), lambda i: (i, 0)),
            pl.BlockSpec((G, 1, cin), lambda i: (0, 0, 0)),
            pl.BlockSpec((G, 1, cin), lambda i: (0, 0, 0)),
            pl.BlockSpec((1, cin), lambda i: (0, 0)),
            pl.BlockSpec((1, cin), lambda i: (0, 0)),
            pl.BlockSpec((cin, cout), lambda i: (0, 0)),
            pl.BlockSpec((1, cout), lambda i: (0, 0)),
        ],
        out_specs=[
            pl.BlockSpec((RB, cout), lambda i: (i, 0)),
            pl.BlockSpec((1, 1, cout), lambda i: (i, 0, 0)),
            pl.BlockSpec((1, 1, cout), lambda i: (i, 0, 0)),
        ],
        out_shape=[
            jax.ShapeDtypeStruct((RTOT, cout), jnp.float32),
            jax.ShapeDtypeStruct((G, 1, cout), jnp.float32),
            jax.ShapeDtypeStruct((G, 1, cout), jnp.float32),
        ],
    )(a, ps, pq, gr, ber, wp, br)


def _fin_body(a_ref, ps_ref, pq_ref, g_ref, be_ref, o_ref):
    cnt = np.float32(RTOT)
    mean = jnp.sum(ps_ref[...], axis=0) / cnt
    ex2 = jnp.sum(pq_ref[...], axis=0) / cnt
    var = ex2 - mean * mean
    rstd = lax.rsqrt(var + EPSF)
    scale = g_ref[...] * rstd
    shift = be_ref[...] - mean * scale
    h = jnp.maximum(a_ref[...] * scale + shift, 0.0)
    o_ref[...] = jnp.max(h.reshape(RB // K, K, 64), axis=1)


def _mlp_fin(a, ps, pq, gr, ber):
    return pl.pallas_call(
        _fin_body,
        grid=(G,),
        in_specs=[
            pl.BlockSpec((RB, 64), lambda i: (i, 0)),
            pl.BlockSpec((G, 1, 64), lambda i: (0, 0, 0)),
            pl.BlockSpec((G, 1, 64), lambda i: (0, 0, 0)),
            pl.BlockSpec((1, 64), lambda i: (0, 0)),
            pl.BlockSpec((1, 64), lambda i: (0, 0)),
        ],
        out_specs=pl.BlockSpec((RB // K, 64), lambda i: (i, 0)),
        out_shape=jax.ShapeDtypeStruct((B * S, 64), jnp.float32),
    )(a, ps, pq, gr, ber)


def kernel(xyzs, feats, W1, b1, g1, be1, W2, b2, g2, be2, W3, b3, g3, be3):
    xs = xyzs[:, :, 0]
    ys = xyzs[:, :, 1]
    zs = xyzs[:, :, 2]

    idxs = _fps(xs, ys, zs)
    centers = jnp.take_along_axis(xyzs, idxs[..., None], axis=1)

    idx = _ball_query_run(centers[..., 0:1], centers[..., 1:2],
                          centers[..., 2:3], xs, ys, zs)

    table = jnp.concatenate(
        [xyzs, feats, jnp.zeros((B, N, DPAD - 19), jnp.float32)], axis=-1
    ).reshape(B * N, DPAD)
    boff = (jnp.arange(B, dtype=jnp.int32) * N)[:, None, None]
    flat_idx = (jnp.clip(idx, 0, N - 1) + boff).reshape(RTOT)
    gathered = _sc_gather(table, flat_idx)

    mask = (idx != -1).astype(jnp.float32).reshape(RTOT, 1)
    crep = jnp.pad(
        jnp.repeat(centers.reshape(B * S, 3), K, axis=0), ((0, 0), (0, 5)))

    w1p = jnp.pad(W1.T, ((0, DPAD - 19), (0, 0)))
    wx = jnp.pad(W1.T[:3], ((0, 5), (0, 0)))
    a1, ps1, pq1 = _mlp1(gathered, crep, mask, w1p, wx, b1.reshape(1, 32))

    a2, ps2, pq2 = _mlp_mid(a1, ps1, pq1, g1.reshape(1, 32), be1.reshape(1, 32),
                            W2.T, b2.reshape(1, 32), 32)
    a3, ps3, pq3 = _mlp_mid(a2, ps2, pq2, g2.reshape(1, 32), be2.reshape(1, 32),
                            W3.T, b3.reshape(1, 64), 64)
    cf = _mlp_fin(a3, ps3, pq3, g3.reshape(1, 64), be3.reshape(1, 64))

    return centers, cf.reshape(B, S, 64)


# speedup vs baseline: 13.7216x; 1.1216x over previous
import functools

import jax
import jax.numpy as jnp
import numpy as np
from jax import lax
from jax.experimental import pallas as pl
from jax.experimental.pallas import tpu as pltpu
from jax.experimental.pallas import tpu_sc as plsc

B = 8
N = 4096
S = 1024
K = 32
RTOT = B * S * K
R2 = np.float32(0.15) * np.float32(0.15)
EPSF = np.float32(1e-5)
RB = 4096
G = RTOT // RB
SB = 128
DPAD = 32


def _fps_body(xs_ref, ys_ref, zs_ref, idx_ref, mind_ref):
    X = xs_ref[...]
    Y = ys_ref[...]
    Z = zs_ref[...]
    lane = lax.broadcasted_iota(jnp.int32, (B, N), 1)
    mind_ref[...] = jnp.full((B, N), jnp.inf, dtype=jnp.float32)

    def body(i, carry):
        lx, ly, lz, acc = carry
        dx = X - lx
        dy = Y - ly
        dz = Z - lz
        d2 = dx * dx + dy * dy + dz * dz
        mind = jnp.minimum(mind_ref[...], d2)
        mind_ref[...] = mind
        m = jnp.max(mind, axis=1, keepdims=True)
        nxt = jnp.min(jnp.where(mind == m, lane, N), axis=1, keepdims=True)
        oh = lane == nxt
        lx = jnp.sum(jnp.where(oh, X, 0.0), axis=1, keepdims=True)
        ly = jnp.sum(jnp.where(oh, Y, 0.0), axis=1, keepdims=True)
        lz = jnp.sum(jnp.where(oh, Z, 0.0), axis=1, keepdims=True)
        si = lax.broadcasted_iota(jnp.int32, (B, S), 1)
        acc = jnp.where(si == i, nxt, acc)
        return lx, ly, lz, acc

    lx0 = X[:, 0:1]
    ly0 = Y[:, 0:1]
    lz0 = Z[:, 0:1]
    acc0 = jnp.zeros((B, S), dtype=jnp.int32)
    _, _, _, acc = lax.fori_loop(1, S, body, (lx0, ly0, lz0, acc0))
    idx_ref[...] = acc


def _fps(xs, ys, zs):
    return pl.pallas_call(
        _fps_body,
        out_shape=jax.ShapeDtypeStruct((B, S), jnp.int32),
        scratch_shapes=[pltpu.VMEM((B, N), jnp.float32)],
    )(xs, ys, zs)


NCH = N // 16


def _bq_body(cx_ref, cy_ref, cz_ref, xs_ref, ys_ref, zs_ref, p_ref, idx_ref):
    cx = cx_ref[0]
    cy = cy_ref[0]
    cz = cz_ref[0]
    X = xs_ref[0]
    Y = ys_ref[0]
    Z = zs_ref[0]
    dx = cx - X
    dy = cy - Y
    dz = cz - Z
    d2 = dx * dx + dy * dy + dz * dz
    within = d2 < R2
    wf = jnp.dot(within.astype(jnp.bfloat16), p_ref[...],
                 preferred_element_type=jnp.float32)
    W = wf.astype(jnp.int32)
    chunk = lax.broadcasted_iota(jnp.int32, (SB, NCH), 1)
    for k in range(K):
        key = jnp.where(W != 0, chunk, NCH)
        mch = jnp.min(key, axis=1, keepdims=True)
        eq = chunk == mch
        w_sel = jnp.sum(jnp.where(eq, W, 0), axis=1, keepdims=True)
        low = w_sel & -w_sel
        pos = lax.shift_right_logical(
            lax.bitcast_convert_type(low.astype(jnp.float32), jnp.int32), 23
        ) - 127
        idx_k = mch * 16 + pos
        idx_ref[0, :, k : k + 1] = jnp.where(mch < NCH, idx_k, -1)
        W = jnp.where(eq, w_sel ^ low, W)


def _ball_query_run(cx, cy, cz, xs, ys, zs, packmat):
    spec_c = pl.BlockSpec((1, SB, 1), lambda b, s: (b, s, 0))
    spec_x = pl.BlockSpec((1, 1, N), lambda b, s: (b, 0, 0))
    spec_p = pl.BlockSpec((N, NCH), lambda b, s: (0, 0))
    return pl.pallas_call(
        _bq_body,
        grid=(B, S // SB),
        in_specs=[spec_c, spec_c, spec_c, spec_x, spec_x, spec_x, spec_p],
        out_specs=pl.BlockSpec((1, SB, K), lambda b, s: (b, s, 0)),
        out_shape=jax.ShapeDtypeStruct((B, S, K), jnp.int32),
    )(cx, cy, cz, xs[:, None, :], ys[:, None, :], zs[:, None, :], packmat)


def _sc_gather(table, flat_idx):
    info = plsc.get_sparse_core_info()
    NC, NS = info.num_cores, info.num_subcores
    NW = NC * NS
    per_w = RTOT // NW
    CH = 128
    OUT_CH = 1024
    n_outer = per_w // OUT_CH
    n_inner = OUT_CH // CH
    idx2d = flat_idx.reshape(RTOT // CH, CH)
    mesh = plsc.VectorSubcoreMesh(core_axis_name="c", subcore_axis_name="s")

    @functools.partial(
        pl.kernel,
        out_type=jax.ShapeDtypeStruct((RTOT, DPAD), jnp.float32),
        mesh=mesh,
        scratch_types=[
            pltpu.VMEM((n_inner, CH), jnp.int32),
            pltpu.VMEM((OUT_CH, DPAD), jnp.float32),
            pltpu.SemaphoreType.DMA,
        ],
        compiler_params=pltpu.CompilerParams(use_tc_tiling_on_sc=False),
    )
    def k(table_hbm, idx_hbm, out_hbm, idx_v, rows_v, sem):
        wid = lax.axis_index("s") * NC + lax.axis_index("c")
        row0 = wid * (per_w // CH)

        def outer(g, _):
            pltpu.sync_copy(idx_hbm.at[pl.ds(row0 + g * n_inner, n_inner)], idx_v)
            copies = []
            for j in range(n_inner):
                copies.append(pltpu.async_copy(
                    table_hbm.at[idx_v.at[j]],
                    rows_v.at[pl.ds(j * CH, CH)],
                    sem,
                ))
            for c in copies:
                c.wait()
            pltpu.sync_copy(
                rows_v, out_hbm.at[pl.ds(wid * per_w + g * OUT_CH, OUT_CH)])
            return 0

        lax.fori_loop(0, n_outer, outer, 0)

    return k(table, idx2d)


def _m1_body(g_ref, c_ref, m_ref, w_ref, wx_ref, b_ref, a_ref, ps_ref, pq_ref):
    gv = g_ref[...] * m_ref[...]
    cm = c_ref[...] * m_ref[...]
    a = (jnp.dot(gv, w_ref[...], preferred_element_type=jnp.float32)
         - jnp.dot(cm, wx_ref[...], preferred_element_type=jnp.float32)
         + b_ref[...])
    a_ref[...] = a
    ps_ref[0] = jnp.sum(a, axis=0, keepdims=True)
    pq_ref[0] = jnp.sum(a * a, axis=0, keepdims=True)


def _mlp1(gathered, crep, mask, w1p, wx, b1r):
    return pl.pallas_call(
        _m1_body,
        grid=(G,),
        in_specs=[
            pl.BlockSpec((RB, DPAD), lambda i: (i, 0)),
            pl.BlockSpec((RB, 8), lambda i: (i, 0)),
            pl.BlockSpec((RB, 1), lambda i: (i, 0)),
            pl.BlockSpec((DPAD, 32), lambda i: (0, 0)),
            pl.BlockSpec((8, 32), lambda i: (0, 0)),
            pl.BlockSpec((1, 32), lambda i: (0, 0)),
        ],
        out_specs=[
            pl.BlockSpec((RB, 32), lambda i: (i, 0)),
            pl.BlockSpec((1, 1, 32), lambda i: (i, 0, 0)),
            pl.BlockSpec((1, 1, 32), lambda i: (i, 0, 0)),
        ],
        out_shape=[
            jax.ShapeDtypeStruct((RTOT, 32), jnp.float32),
            jax.ShapeDtypeStruct((G, 1, 32), jnp.float32),
            jax.ShapeDtypeStruct((G, 1, 32), jnp.float32),
        ],
    )(gathered, crep, mask, w1p, wx, b1r)


def _mid_body(a_ref, ps_ref, pq_ref, g_ref, be_ref, w_ref, b_ref,
              o_ref, ops_ref, opq_ref):
    cnt = np.float32(RTOT)
    mean = jnp.sum(ps_ref[...], axis=0) / cnt
    ex2 = jnp.sum(pq_ref[...], axis=0) / cnt
    var = ex2 - mean * mean
    rstd = lax.rsqrt(var + EPSF)
    scale = g_ref[...] * rstd
    shift = be_ref[...] - mean * scale
    h = jnp.maximum(a_ref[...] * scale + shift, 0.0)
    o = jnp.dot(h, w_ref[...], preferred_element_type=jnp.float32) + b_ref[...]
    o_ref[...] = o
    ops_ref[0] = jnp.sum(o, axis=0, keepdims=True)
    opq_ref[0] = jnp.sum(o * o, axis=0, keepdims=True)


def _mlp_mid(a, ps, pq, gr, ber, wp, br, cout):
    cin = a.shape[1]
    return pl.pallas_call(
        _mid_body,
        grid=(G,),
        in_specs=[
            pl.BlockSpec((RB, cin), lambda i: (i, 0)),
            pl.BlockSpec((G, 1, cin), lambda i: (0, 0, 0)),
            pl.BlockSpec((G, 1, cin), lambda i: (0, 0, 0)),
            pl.BlockSpec((1, cin), lambda i: (0, 0)),
            pl.BlockSpec((1, cin), lambda i: (0, 0)),
            pl.BlockSpec((cin, cout), lambda i: (0, 0)),
            pl.BlockSpec((1, cout), lambda i: (0, 0)),
        ],
        out_specs=[
            pl.BlockSpec((RB, cout), lambda i: (i, 0)),
            pl.BlockSpec((1, 1, cout), lambda i: (i, 0, 0)),
            pl.BlockSpec((1, 1, cout), lambda i: (i, 0, 0)),
        ],
        out_shape=[
            jax.ShapeDtypeStruct((RTOT, cout), jnp.float32),
            jax.ShapeDtypeStruct((G, 1, cout), jnp.float32),
            jax.ShapeDtypeStruct((G, 1, cout), jnp.float32),
        ],
    )(a, ps, pq, gr, ber, wp, br)


def _fin_body(a_ref, ps_ref, pq_ref, g_ref, be_ref, o_ref):
    cnt = np.float32(RTOT)
    mean = jnp.sum(ps_ref[...], axis=0) / cnt
    ex2 = jnp.sum(pq_ref[...], axis=0) / cnt
    var = ex2 - mean * mean
    rstd = lax.rsqrt(var + EPSF)
    scale = g_ref[...] * rstd
    shift = be_ref[...] - mean * scale
    h = jnp.maximum(a_ref[...] * scale + shift, 0.0)
    o_ref[...] = jnp.max(h.reshape(RB // K, K, 64), axis=1)


def _mlp_fin(a, ps, pq, gr, ber):
    return pl.pallas_call(
        _fin_body,
        grid=(G,),
        in_specs=[
            pl.BlockSpec((RB, 64), lambda i: (i, 0)),
            pl.BlockSpec((G, 1, 64), lambda i: (0, 0, 0)),
            pl.BlockSpec((G, 1, 64), lambda i: (0, 0, 0)),
            pl.BlockSpec((1, 64), lambda i: (0, 0)),
            pl.BlockSpec((1, 64), lambda i: (0, 0)),
        ],
        out_specs=pl.BlockSpec((RB // K, 64), lambda i: (i, 0)),
        out_shape=jax.ShapeDtypeStruct((B * S, 64), jnp.float32),
    )(a, ps, pq, gr, ber)


def kernel(xyzs, feats, W1, b1, g1, be1, W2, b2, g2, be2, W3, b3, g3, be3):
    xs = xyzs[:, :, 0]
    ys = xyzs[:, :, 1]
    zs = xyzs[:, :, 2]

    idxs = _fps(xs, ys, zs)
    centers = jnp.take_along_axis(xyzs, idxs[..., None], axis=1)

    pmat = (np.left_shift(1, np.arange(N) % 16)[:, None]
            * (np.arange(N)[:, None] // 16 == np.arange(NCH)[None, :]))
    packmat = jnp.asarray(pmat, dtype=jnp.bfloat16)
    idx = _ball_query_run(centers[..., 0:1], centers[..., 1:2],
                          centers[..., 2:3], xs, ys, zs, packmat)

    table = jnp.concatenate(
        [xyzs, feats, jnp.zeros((B, N, DPAD - 19), jnp.float32)], axis=-1
    ).reshape(B * N, DPAD)
    boff = (jnp.arange(B, dtype=jnp.int32) * N)[:, None, None]
    flat_idx = (jnp.clip(idx, 0, N - 1) + boff).reshape(RTOT)
    gathered = _sc_gather(table, flat_idx)

    mask = (idx != -1).astype(jnp.float32).reshape(RTOT, 1)
    crep = jnp.pad(
        jnp.repeat(centers.reshape(B * S, 3), K, axis=0), ((0, 0), (0, 5)))

    w1p = jnp.pad(W1.T, ((0, DPAD - 19), (0, 0)))
    wx = jnp.pad(W1.T[:3], ((0, 5), (0, 0)))
    a1, ps1, pq1 = _mlp1(gathered, crep, mask, w1p, wx, b1.reshape(1, 32))

    a2, ps2, pq2 = _mlp_mid(a1, ps1, pq1, g1.reshape(1, 32), be1.reshape(1, 32),
                            W2.T, b2.reshape(1, 32), 32)
    a3, ps3, pq3 = _mlp_mid(a2, ps2, pq2, g2.reshape(1, 32), be2.reshape(1, 32),
                            W3.T, b3.reshape(1, 64), 64)
    cf = _mlp_fin(a3, ps3, pq3, g3.reshape(1, 64), be3.reshape(1, 64))

    return centers, cf.reshape(B, S, 64)


# speedup vs baseline: 14.9581x; 1.0901x over previous
import functools

import jax
import jax.numpy as jnp
import numpy as np
from jax import lax
from jax.experimental import pallas as pl
from jax.experimental.pallas import tpu as pltpu
from jax.experimental.pallas import tpu_sc as plsc

B = 8
N = 4096
S = 1024
K = 32
RTOT = B * S * K
R2 = np.float32(0.15) * np.float32(0.15)
EPSF = np.float32(1e-5)
RB = 4096
G = RTOT // RB
SB = 128
DPAD = 32


def _fps_body(xs_ref, ys_ref, zs_ref, idx_ref, mind_ref):
    X = xs_ref[...]
    Y = ys_ref[...]
    Z = zs_ref[...]
    lane = lax.broadcasted_iota(jnp.int32, (B, N), 1)
    mind_ref[...] = jnp.full((B, N), jnp.inf, dtype=jnp.float32)

    def body(i, carry):
        lx, ly, lz, acc = carry
        dx = X - lx
        dy = Y - ly
        dz = Z - lz
        d2 = dx * dx + dy * dy + dz * dz
        mind = jnp.minimum(mind_ref[...], d2)
        mind_ref[...] = mind
        m = jnp.max(mind, axis=1, keepdims=True)
        nxt = jnp.min(jnp.where(mind == m, lane, N), axis=1, keepdims=True)
        oh = lane == nxt
        lx = jnp.sum(jnp.where(oh, X, 0.0), axis=1, keepdims=True)
        ly = jnp.sum(jnp.where(oh, Y, 0.0), axis=1, keepdims=True)
        lz = jnp.sum(jnp.where(oh, Z, 0.0), axis=1, keepdims=True)
        si = lax.broadcasted_iota(jnp.int32, (B, S), 1)
        acc = jnp.where(si == i, nxt, acc)
        return lx, ly, lz, acc

    lx0 = X[:, 0:1]
    ly0 = Y[:, 0:1]
    lz0 = Z[:, 0:1]
    acc0 = jnp.zeros((B, S), dtype=jnp.int32)
    _, _, _, acc = lax.fori_loop(1, S, body, (lx0, ly0, lz0, acc0))
    idx_ref[...] = acc


def _fps(xs, ys, zs):
    return pl.pallas_call(
        _fps_body,
        out_shape=jax.ShapeDtypeStruct((B, S), jnp.int32),
        scratch_shapes=[pltpu.VMEM((B, N), jnp.float32)],
    )(xs, ys, zs)


NCH = N // 16


def _bq_body(cx_ref, cy_ref, cz_ref, xs_ref, ys_ref, zs_ref, p_ref, idx_ref):
    cx = cx_ref[0]
    cy = cy_ref[0]
    cz = cz_ref[0]
    X = xs_ref[0]
    Y = ys_ref[0]
    Z = zs_ref[0]
    dx = cx - X
    dy = cy - Y
    dz = cz - Z
    d2 = dx * dx + dy * dy + dz * dz
    within = d2 < R2
    wf = jnp.dot(within.astype(jnp.bfloat16), p_ref[...],
                 preferred_element_type=jnp.float32)
    W = wf.astype(jnp.int32)
    chunk = lax.broadcasted_iota(jnp.int32, (SB, NCH), 1)
    BIGK = jnp.int32(1 << 30)
    key2 = jnp.where(W != 0, (chunk << 16) | W, BIGK)
    for k in range(K):
        m2 = jnp.min(key2, axis=1, keepdims=True)
        w_sel = m2 & 0xFFFF
        mch = lax.shift_right_logical(m2, 16)
        low = w_sel & -w_sel
        pos = lax.shift_right_logical(
            lax.bitcast_convert_type(low.astype(jnp.float32), jnp.int32), 23
        ) - 127
        idx_k = mch * 16 + pos
        idx_ref[0, :, k : k + 1] = jnp.where(m2 < BIGK, idx_k, -1)
        newkey = jnp.where(w_sel == low, BIGK, m2 ^ low)
        key2 = jnp.where(key2 == m2, newkey, key2)


def _ball_query_run(cx, cy, cz, xs, ys, zs, packmat):
    spec_c = pl.BlockSpec((1, SB, 1), lambda b, s: (b, s, 0))
    spec_x = pl.BlockSpec((1, 1, N), lambda b, s: (b, 0, 0))
    spec_p = pl.BlockSpec((N, NCH), lambda b, s: (0, 0))
    return pl.pallas_call(
        _bq_body,
        grid=(B, S // SB),
        in_specs=[spec_c, spec_c, spec_c, spec_x, spec_x, spec_x, spec_p],
        out_specs=pl.BlockSpec((1, SB, K), lambda b, s: (b, s, 0)),
        out_shape=jax.ShapeDtypeStruct((B, S, K), jnp.int32),
    )(cx, cy, cz, xs[:, None, :], ys[:, None, :], zs[:, None, :], packmat)


def _sc_gather(table, flat_idx):
    info = plsc.get_sparse_core_info()
    NC, NS = info.num_cores, info.num_subcores
    NW = NC * NS
    per_w = RTOT // NW
    CH = 128
    OUT_CH = 1024
    n_outer = per_w // OUT_CH
    n_inner = OUT_CH // CH
    idx2d = flat_idx.reshape(RTOT // CH, CH)
    mesh = plsc.VectorSubcoreMesh(core_axis_name="c", subcore_axis_name="s")

    @functools.partial(
        pl.kernel,
        out_type=jax.ShapeDtypeStruct((RTOT, DPAD), jnp.float32),
        mesh=mesh,
        scratch_types=[
            pltpu.VMEM((n_inner, CH), jnp.int32),
            pltpu.VMEM((OUT_CH, DPAD), jnp.float32),
            pltpu.SemaphoreType.DMA,
        ],
        compiler_params=pltpu.CompilerParams(use_tc_tiling_on_sc=False),
    )
    def k(table_hbm, idx_hbm, out_hbm, idx_v, rows_v, sem):
        wid = lax.axis_index("s") * NC + lax.axis_index("c")
        row0 = wid * (per_w // CH)

        def outer(g, _):
            pltpu.sync_copy(idx_hbm.at[pl.ds(row0 + g * n_inner, n_inner)], idx_v)
            copies = []
            for j in range(n_inner):
                copies.append(pltpu.async_copy(
                    table_hbm.at[idx_v.at[j]],
                    rows_v.at[pl.ds(j * CH, CH)],
                    sem,
                ))
            for c in copies:
                c.wait()
            pltpu.sync_copy(
                rows_v, out_hbm.at[pl.ds(wid * per_w + g * OUT_CH, OUT_CH)])
            return 0

        lax.fori_loop(0, n_outer, outer, 0)

    return k(table, idx2d)


def _m1_body(g_ref, c_ref, m_ref, w_ref, wx_ref, b_ref, a_ref, ps_ref, pq_ref):
    gv = g_ref[...] * m_ref[...]
    cm = c_ref[...] * m_ref[...]
    a = (jnp.dot(gv, w_ref[...], preferred_element_type=jnp.float32)
         - jnp.dot(cm, wx_ref[...], preferred_element_type=jnp.float32)
         + b_ref[...])
    a_ref[...] = a
    ps_ref[0] = jnp.sum(a, axis=0, keepdims=True)
    pq_ref[0] = jnp.sum(a * a, axis=0, keepdims=True)


def _mlp1(gathered, crep, mask, w1p, wx, b1r):
    return pl.pallas_call(
        _m1_body,
        grid=(G,),
        in_specs=[
            pl.BlockSpec((RB, DPAD), lambda i: (i, 0)),
            pl.BlockSpec((RB, 8), lambda i: (i, 0)),
            pl.BlockSpec((RB, 1), lambda i: (i, 0)),
            pl.BlockSpec((DPAD, 32), lambda i: (0, 0)),
            pl.BlockSpec((8, 32), lambda i: (0, 0)),
            pl.BlockSpec((1, 32), lambda i: (0, 0)),
        ],
        out_specs=[
            pl.BlockSpec((RB, 32), lambda i: (i, 0)),
            pl.BlockSpec((1, 1, 32), lambda i: (i, 0, 0)),
            pl.BlockSpec((1, 1, 32), lambda i: (i, 0, 0)),
        ],
        out_shape=[
            jax.ShapeDtypeStruct((RTOT, 32), jnp.float32),
            jax.ShapeDtypeStruct((G, 1, 32), jnp.float32),
            jax.ShapeDtypeStruct((G, 1, 32), jnp.float32),
        ],
    )(gathered, crep, mask, w1p, wx, b1r)


def _mid_body(a_ref, ps_ref, pq_ref, g_ref, be_ref, w_ref, b_ref,
              o_ref, ops_ref, opq_ref):
    cnt = np.float32(RTOT)
    mean = jnp.sum(ps_ref[...], axis=0) / cnt
    ex2 = jnp.sum(pq_ref[...], axis=0) / cnt
    var = ex2 - mean * mean
    rstd = lax.rsqrt(var + EPSF)
    scale = g_ref[...] * rstd
    shift = be_ref[...] - mean * scale
    h = jnp.maximum(a_ref[...] * scale + shift, 0.0)
    o = jnp.dot(h, w_ref[...], preferred_element_type=jnp.float32) + b_ref[...]
    o_ref[...] = o
    ops_ref[0] = jnp.sum(o, axis=0, keepdims=True)
    opq_ref[0] = jnp.sum(o * o, axis=0, keepdims=True)


def _mlp_mid(a, ps, pq, gr, ber, wp, br, cout):
    cin = a.shape[1]
    return pl.pallas_call(
        _mid_body,
        grid=(G,),
        in_specs=[
            pl.BlockSpec((RB, cin), lambda i: (i, 0)),
            pl.BlockSpec((G, 1, cin), lambda i: (0, 0, 0)),
            pl.BlockSpec((G, 1, cin), lambda i: (0, 0, 0)),
            pl.BlockSpec((1, cin), lambda i: (0, 0)),
            pl.BlockSpec((1, cin), lambda i: (0, 0)),
            pl.BlockSpec((cin, cout), lambda i: (0, 0)),
            pl.BlockSpec((1, cout), lambda i: (0, 0)),
        ],
        out_specs=[
            pl.BlockSpec((RB, cout), lambda i: (i, 0)),
            pl.BlockSpec((1, 1, cout), lambda i: (i, 0, 0)),
            pl.BlockSpec((1, 1, cout), lambda i: (i, 0, 0)),
        ],
        out_shape=[
            jax.ShapeDtypeStruct((RTOT, cout), jnp.float32),
            jax.ShapeDtypeStruct((G, 1, cout), jnp.float32),
            jax.ShapeDtypeStruct((G, 1, cout), jnp.float32),
        ],
    )(a, ps, pq, gr, ber, wp, br)


def _fin_body(a_ref, ps_ref, pq_ref, g_ref, be_ref, o_ref):
    cnt = np.float32(RTOT)
    mean = jnp.sum(ps_ref[...], axis=0) / cnt
    ex2 = jnp.sum(pq_ref[...], axis=0) / cnt
    var = ex2 - mean * mean
    rstd = lax.rsqrt(var + EPSF)
    scale = g_ref[...] * rstd
    shift = be_ref[...] - mean * scale
    h = jnp.maximum(a_ref[...] * scale + shift, 0.0)
    o_ref[...] = jnp.max(h.reshape(RB // K, K, 64), axis=1)


def _mlp_fin(a, ps, pq, gr, ber):
    return pl.pallas_call(
        _fin_body,
        grid=(G,),
        in_specs=[
            pl.BlockSpec((RB, 64), lambda i: (i, 0)),
            pl.BlockSpec((G, 1, 64), lambda i: (0, 0, 0)),
            pl.BlockSpec((G, 1, 64), lambda i: (0, 0, 0)),
            pl.BlockSpec((1, 64), lambda i: (0, 0)),
            pl.BlockSpec((1, 64), lambda i: (0, 0)),
        ],
        out_specs=pl.BlockSpec((RB // K, 64), lambda i: (i, 0)),
        out_shape=jax.ShapeDtypeStruct((B * S, 64), jnp.float32),
    )(a, ps, pq, gr, ber)


def kernel(xyzs, feats, W1, b1, g1, be1, W2, b2, g2, be2, W3, b3, g3, be3):
    xs = xyzs[:, :, 0]
    ys = xyzs[:, :, 1]
    zs = xyzs[:, :, 2]

    idxs = _fps(xs, ys, zs)
    centers = jnp.take_along_axis(xyzs, idxs[..., None], axis=1)

    pmat = (np.left_shift(1, np.arange(N) % 16)[:, None]
            * (np.arange(N)[:, None] // 16 == np.arange(NCH)[None, :]))
    packmat = jnp.asarray(pmat, dtype=jnp.bfloat16)
    idx = _ball_query_run(centers[..., 0:1], centers[..., 1:2],
                          centers[..., 2:3], xs, ys, zs, packmat)

    table = jnp.concatenate(
        [xyzs, feats, jnp.zeros((B, N, DPAD - 19), jnp.float32)], axis=-1
    ).reshape(B * N, DPAD)
    boff = (jnp.arange(B, dtype=jnp.int32) * N)[:, None, None]
    flat_idx = (jnp.clip(idx, 0, N - 1) + boff).reshape(RTOT)
    gathered = _sc_gather(table, flat_idx)

    mask = (idx != -1).astype(jnp.float32).reshape(RTOT, 1)
    crep = jnp.pad(
        jnp.repeat(centers.reshape(B * S, 3), K, axis=0), ((0, 0), (0, 5)))

    w1p = jnp.pad(W1.T, ((0, DPAD - 19), (0, 0)))
    wx = jnp.pad(W1.T[:3], ((0, 5), (0, 0)))
    a1, ps1, pq1 = _mlp1(gathered, crep, mask, w1p, wx, b1.reshape(1, 32))

    a2, ps2, pq2 = _mlp_mid(a1, ps1, pq1, g1.reshape(1, 32), be1.reshape(1, 32),
                            W2.T, b2.reshape(1, 32), 32)
    a3, ps3, pq3 = _mlp_mid(a2, ps2, pq2, g2.reshape(1, 32), be2.reshape(1, 32),
                            W3.T, b3.reshape(1, 64), 64)
    cf = _mlp_fin(a3, ps3, pq3, g3.reshape(1, 64), be3.reshape(1, 64))

    return centers, cf.reshape(B, S, 64)


# speedup vs baseline: 16.7934x; 1.1227x over previous
import functools

import jax
import jax.numpy as jnp
import numpy as np
from jax import lax
from jax.experimental import pallas as pl
from jax.experimental.pallas import tpu as pltpu
from jax.experimental.pallas import tpu_sc as plsc

B = 8
N = 4096
S = 1024
K = 32
RTOT = B * S * K
R2 = np.float32(0.15) * np.float32(0.15)
EPSF = np.float32(1e-5)
RB = 4096
G = RTOT // RB
SB = 128
DPAD = 32


def _fps_body(xs_ref, ys_ref, zs_ref, idx_ref, mind_ref):
    X = xs_ref[...]
    Y = ys_ref[...]
    Z = zs_ref[...]
    lane = lax.broadcasted_iota(jnp.int32, (B, N), 1)
    mind_ref[...] = jnp.full((B, N), jnp.inf, dtype=jnp.float32)

    def body(i, carry):
        lx, ly, lz, acc = carry
        dx = X - lx
        dy = Y - ly
        dz = Z - lz
        d2 = dx * dx + dy * dy + dz * dz
        mind = jnp.minimum(mind_ref[...], d2)
        mind_ref[...] = mind
        m = jnp.max(mind, axis=1, keepdims=True)
        nxt = jnp.min(jnp.where(mind == m, lane, N), axis=1, keepdims=True)
        oh = lane == nxt
        lx = jnp.sum(jnp.where(oh, X, 0.0), axis=1, keepdims=True)
        ly = jnp.sum(jnp.where(oh, Y, 0.0), axis=1, keepdims=True)
        lz = jnp.sum(jnp.where(oh, Z, 0.0), axis=1, keepdims=True)
        si = lax.broadcasted_iota(jnp.int32, (B, S), 1)
        acc = jnp.where(si == i, nxt, acc)
        return lx, ly, lz, acc

    lx0 = X[:, 0:1]
    ly0 = Y[:, 0:1]
    lz0 = Z[:, 0:1]
    acc0 = jnp.zeros((B, S), dtype=jnp.int32)
    _, _, _, acc = lax.fori_loop(1, S, body, (lx0, ly0, lz0, acc0))
    idx_ref[...] = acc


def _fps(xs, ys, zs):
    return pl.pallas_call(
        _fps_body,
        out_shape=jax.ShapeDtypeStruct((B, S), jnp.int32),
        scratch_shapes=[pltpu.VMEM((B, N), jnp.float32)],
    )(xs, ys, zs)


NCH = N // 16


def _bq_body(cx_ref, cy_ref, cz_ref, xs_ref, ys_ref, zs_ref, p_ref, idx_ref):
    cx = cx_ref[0]
    cy = cy_ref[0]
    cz = cz_ref[0]
    X = xs_ref[0]
    Y = ys_ref[0]
    Z = zs_ref[0]
    dx = cx - X
    dy = cy - Y
    dz = cz - Z
    d2 = dx * dx + dy * dy + dz * dz
    within = d2 < R2
    wf = jnp.dot(within.astype(jnp.bfloat16), p_ref[...],
                 preferred_element_type=jnp.float32)
    W = wf.astype(jnp.int32)
    chunk = lax.broadcasted_iota(jnp.int32, (SB, NCH), 1)
    BIGK = jnp.int32(1 << 30)
    key2 = jnp.where(W != 0, (chunk << 16) | W, BIGK)
    for k in range(K):
        m2 = jnp.min(key2, axis=1, keepdims=True)
        w_sel = m2 & 0xFFFF
        mch = lax.shift_right_logical(m2, 16)
        low = w_sel & -w_sel
        pos = lax.shift_right_logical(
            lax.bitcast_convert_type(low.astype(jnp.float32), jnp.int32), 23
        ) - 127
        idx_k = mch * 16 + pos
        idx_ref[0, :, k : k + 1] = jnp.where(m2 < BIGK, idx_k, -1)
        newkey = jnp.where(w_sel == low, BIGK, m2 ^ low)
        key2 = jnp.where(key2 == m2, newkey, key2)


def _ball_query_run(cx, cy, cz, xs, ys, zs, packmat):
    spec_c = pl.BlockSpec((1, SB, 1), lambda b, s: (b, s, 0))
    spec_x = pl.BlockSpec((1, 1, N), lambda b, s: (b, 0, 0))
    spec_p = pl.BlockSpec((N, NCH), lambda b, s: (0, 0))
    return pl.pallas_call(
        _bq_body,
        grid=(B, S // SB),
        in_specs=[spec_c, spec_c, spec_c, spec_x, spec_x, spec_x, spec_p],
        out_specs=pl.BlockSpec((1, SB, K), lambda b, s: (b, s, 0)),
        out_shape=jax.ShapeDtypeStruct((B, S, K), jnp.int32),
    )(cx, cy, cz, xs[:, None, :], ys[:, None, :], zs[:, None, :], packmat)


def _sc_gather(table, flat_idx):
    info = plsc.get_sparse_core_info()
    NC, NS = info.num_cores, info.num_subcores
    NW = NC * NS
    per_w = RTOT // NW
    CH = 128
    OUT_CH = 1024
    n_outer = per_w // OUT_CH
    n_inner = OUT_CH // CH
    idx2d = flat_idx.reshape(RTOT // CH, CH)
    mesh = plsc.VectorSubcoreMesh(core_axis_name="c", subcore_axis_name="s")

    @functools.partial(
        pl.kernel,
        out_type=jax.ShapeDtypeStruct((RTOT, DPAD), jnp.float32),
        mesh=mesh,
        scratch_types=[
            pltpu.VMEM((n_inner, CH), jnp.int32),
            pltpu.VMEM((OUT_CH, DPAD), jnp.float32),
            pltpu.SemaphoreType.DMA,
        ],
        compiler_params=pltpu.CompilerParams(use_tc_tiling_on_sc=False),
    )
    def k(table_hbm, idx_hbm, out_hbm, idx_v, rows_v, sem):
        wid = lax.axis_index("s") * NC + lax.axis_index("c")
        row0 = wid * (per_w // CH)

        def outer(g, _):
            pltpu.sync_copy(idx_hbm.at[pl.ds(row0 + g * n_inner, n_inner)], idx_v)
            copies = []
            for j in range(n_inner):
                copies.append(pltpu.async_copy(
                    table_hbm.at[idx_v.at[j]],
                    rows_v.at[pl.ds(j * CH, CH)],
                    sem,
                ))
            for c in copies:
                c.wait()
            pltpu.sync_copy(
                rows_v, out_hbm.at[pl.ds(wid * per_w + g * OUT_CH, OUT_CH)])
            return 0

        lax.fori_loop(0, n_outer, outer, 0)

    return k(table, idx2d)


def _m1_body(g_ref, c_ref, w_ref, wx_ref, b_ref, a_ref, ps_ref, pq_ref):
    g = g_ref[...]
    m = g[:, 19:20]
    crep = jnp.broadcast_to(
        c_ref[...][:, None, :], (RB // K, K, 8)).reshape(RB, 8)
    a = (jnp.dot(g, w_ref[...], preferred_element_type=jnp.float32)
         - jnp.dot(crep * m, wx_ref[...], preferred_element_type=jnp.float32)
         + b_ref[...])
    a_ref[...] = a
    ps_ref[0] = jnp.sum(a, axis=0, keepdims=True)
    pq_ref[0] = jnp.sum(a * a, axis=0, keepdims=True)


def _mlp1(gathered, cpad, w1p, wx, b1r):
    return pl.pallas_call(
        _m1_body,
        grid=(G,),
        in_specs=[
            pl.BlockSpec((RB, DPAD), lambda i: (i, 0)),
            pl.BlockSpec((RB // K, 8), lambda i: (i, 0)),
            pl.BlockSpec((DPAD, 32), lambda i: (0, 0)),
            pl.BlockSpec((8, 32), lambda i: (0, 0)),
            pl.BlockSpec((1, 32), lambda i: (0, 0)),
        ],
        out_specs=[
            pl.BlockSpec((RB, 32), lambda i: (i, 0)),
            pl.BlockSpec((1, 1, 32), lambda i: (i, 0, 0)),
            pl.BlockSpec((1, 1, 32), lambda i: (i, 0, 0)),
        ],
        out_shape=[
            jax.ShapeDtypeStruct((RTOT, 32), jnp.float32),
            jax.ShapeDtypeStruct((G, 1, 32), jnp.float32),
            jax.ShapeDtypeStruct((G, 1, 32), jnp.float32),
        ],
    )(gathered, cpad, w1p, wx, b1r)


def _mid_body(a_ref, ps_ref, pq_ref, g_ref, be_ref, w_ref, b_ref,
              o_ref, ops_ref, opq_ref):
    cnt = np.float32(RTOT)
    mean = jnp.sum(ps_ref[...], axis=0) / cnt
    ex2 = jnp.sum(pq_ref[...], axis=0) / cnt
    var = ex2 - mean * mean
    rstd = lax.rsqrt(var + EPSF)
    scale = g_ref[...] * rstd
    shift = be_ref[...] - mean * scale
    h = jnp.maximum(a_ref[...] * scale + shift, 0.0)
    o = jnp.dot(h, w_ref[...], preferred_element_type=jnp.float32) + b_ref[...]
    o_ref[...] = o
    ops_ref[0] = jnp.sum(o, axis=0, keepdims=True)
    opq_ref[0] = jnp.sum(o * o, axis=0, keepdims=True)


def _mlp_mid(a, ps, pq, gr, ber, wp, br, cout):
    cin = a.shape[1]
    return pl.pallas_call(
        _mid_body,
        grid=(G,),
        in_specs=[
            pl.BlockSpec((RB, cin), lambda i: (i, 0)),
            pl.BlockSpec((G, 1, cin), lambda i: (0, 0, 0)),
            pl.BlockSpec((G, 1, cin), lambda i: (0, 0, 0)),
            pl.BlockSpec((1, cin), lambda i: (0, 0)),
            pl.BlockSpec((1, cin), lambda i: (0, 0)),
            pl.BlockSpec((cin, cout), lambda i: (0, 0)),
            pl.BlockSpec((1, cout), lambda i: (0, 0)),
        ],
        out_specs=[
            pl.BlockSpec((RB, cout), lambda i: (i, 0)),
            pl.BlockSpec((1, 1, cout), lambda i: (i, 0, 0)),
            pl.BlockSpec((1, 1, cout), lambda i: (i, 0, 0)),
        ],
        out_shape=[
            jax.ShapeDtypeStruct((RTOT, cout), jnp.float32),
            jax.ShapeDtypeStruct((G, 1, cout), jnp.float32),
            jax.ShapeDtypeStruct((G, 1, cout), jnp.float32),
        ],
    )(a, ps, pq, gr, ber, wp, br)


def _fin_body(a_ref, ps_ref, pq_ref, g_ref, be_ref, o_ref):
    cnt = np.float32(RTOT)
    mean = jnp.sum(ps_ref[...], axis=0) / cnt
    ex2 = jnp.sum(pq_ref[...], axis=0) / cnt
    var = ex2 - mean * mean
    rstd = lax.rsqrt(var + EPSF)
    scale = g_ref[...] * rstd
    shift = be_ref[...] - mean * scale
    h = jnp.maximum(a_ref[...] * scale + shift, 0.0)
    o_ref[...] = jnp.max(h.reshape(RB // K, K, 64), axis=1)


def _mlp_fin(a, ps, pq, gr, ber):
    return pl.pallas_call(
        _fin_body,
        grid=(G,),
        in_specs=[
            pl.BlockSpec((RB, 64), lambda i: (i, 0)),
            pl.BlockSpec((G, 1, 64), lambda i: (0, 0, 0)),
            pl.BlockSpec((G, 1, 64), lambda i: (0, 0, 0)),
            pl.BlockSpec((1, 64), lambda i: (0, 0)),
            pl.BlockSpec((1, 64), lambda i: (0, 0)),
        ],
        out_specs=pl.BlockSpec((RB // K, 64), lambda i: (i, 0)),
        out_shape=jax.ShapeDtypeStruct((B * S, 64), jnp.float32),
    )(a, ps, pq, gr, ber)


def kernel(xyzs, feats, W1, b1, g1, be1, W2, b2, g2, be2, W3, b3, g3, be3):
    xs = xyzs[:, :, 0]
    ys = xyzs[:, :, 1]
    zs = xyzs[:, :, 2]

    idxs = _fps(xs, ys, zs)
    centers = jnp.take_along_axis(xyzs, idxs[..., None], axis=1)

    pmat = (np.left_shift(1, np.arange(N) % 16)[:, None]
            * (np.arange(N)[:, None] // 16 == np.arange(NCH)[None, :]))
    packmat = jnp.asarray(pmat, dtype=jnp.bfloat16)
    idx = _ball_query_run(centers[..., 0:1], centers[..., 1:2],
                          centers[..., 2:3], xs, ys, zs, packmat)

    table = jnp.concatenate(
        [xyzs, feats, jnp.ones((B, N, 1), jnp.float32),
         jnp.zeros((B, N, DPAD - 20), jnp.float32)], axis=-1
    ).reshape(B * N, DPAD)
    table = jnp.pad(table, ((0, 8), (0, 0)))
    boff = (jnp.arange(B, dtype=jnp.int32) * N)[:, None, None]
    flat_idx = jnp.where(idx >= 0, idx + boff, B * N).reshape(RTOT)
    gathered = _sc_gather(table, flat_idx)

    cpad = jnp.pad(centers.reshape(B * S, 3), ((0, 0), (0, 5)))

    w1p = jnp.pad(W1.T, ((0, DPAD - 19), (0, 0)))
    wx = jnp.pad(W1.T[:3], ((0, 5), (0, 0)))
    a1, ps1, pq1 = _mlp1(gathered, cpad, w1p, wx, b1.reshape(1, 32))

    a2, ps2, pq2 = _mlp_mid(a1, ps1, pq1, g1.reshape(1, 32), be1.reshape(1, 32),
                            W2.T, b2.reshape(1, 32), 32)
    a3, ps3, pq3 = _mlp_mid(a2, ps2, pq2, g2.reshape(1, 32), be2.reshape(1, 32),
                            W3.T, b3.reshape(1, 64), 64)
    cf = _mlp_fin(a3, ps3, pq3, g3.reshape(1, 64), be3.reshape(1, 64))

    return centers, cf.reshape(B, S, 64)


# speedup vs baseline: 16.7981x; 1.0003x over previous
import functools

import jax
import jax.numpy as jnp
import numpy as np
from jax import lax
from jax.experimental import pallas as pl
from jax.experimental.pallas import tpu as pltpu
from jax.experimental.pallas import tpu_sc as plsc

B = 8
N = 4096
S = 1024
K = 32
RTOT = B * S * K
R2 = np.float32(0.15) * np.float32(0.15)
EPSF = np.float32(1e-5)
RB = 4096
G = RTOT // RB
SB = 128
DPAD = 32


def _fps_body(xs_ref, ys_ref, zs_ref, idx_ref, mind_ref):
    X = xs_ref[...]
    Y = ys_ref[...]
    Z = zs_ref[...]
    lane = lax.broadcasted_iota(jnp.int32, (B, N), 1)
    mind_ref[...] = jnp.full((B, N), jnp.inf, dtype=jnp.float32)

    def body(i, carry):
        lx, ly, lz, acc = carry
        dx = X - lx
        dy = Y - ly
        dz = Z - lz
        d2 = dx * dx + dy * dy + dz * dz
        mind = jnp.minimum(mind_ref[...], d2)
        mind_ref[...] = mind
        m = jnp.max(mind, axis=1, keepdims=True)
        nxt = jnp.min(jnp.where(mind == m, lane, N), axis=1, keepdims=True)
        oh = lane == nxt
        lx = jnp.sum(jnp.where(oh, X, 0.0), axis=1, keepdims=True)
        ly = jnp.sum(jnp.where(oh, Y, 0.0), axis=1, keepdims=True)
        lz = jnp.sum(jnp.where(oh, Z, 0.0), axis=1, keepdims=True)
        si = lax.broadcasted_iota(jnp.int32, (B, S), 1)
        acc = jnp.where(si == i, nxt, acc)
        return lx, ly, lz, acc

    lx0 = X[:, 0:1]
    ly0 = Y[:, 0:1]
    lz0 = Z[:, 0:1]
    acc0 = jnp.zeros((B, S), dtype=jnp.int32)
    _, _, _, acc = lax.fori_loop(1, S, body, (lx0, ly0, lz0, acc0))
    idx_ref[...] = acc


def _fps(xs, ys, zs):
    return pl.pallas_call(
        _fps_body,
        out_shape=jax.ShapeDtypeStruct((B, S), jnp.int32),
        scratch_shapes=[pltpu.VMEM((B, N), jnp.float32)],
    )(xs, ys, zs)


NCH = N // 16


def _bq_body(cx_ref, cy_ref, cz_ref, xs_ref, ys_ref, zs_ref, p_ref, idx_ref):
    cx = cx_ref[0]
    cy = cy_ref[0]
    cz = cz_ref[0]
    X = xs_ref[0]
    Y = ys_ref[0]
    Z = zs_ref[0]
    dx = cx - X
    dy = cy - Y
    dz = cz - Z
    d2 = dx * dx + dy * dy + dz * dz
    within = d2 < R2
    wf = jnp.dot(within.astype(jnp.bfloat16), p_ref[...],
                 preferred_element_type=jnp.float32)
    W = wf.astype(jnp.int32)
    chunk = lax.broadcasted_iota(jnp.int32, (SB, NCH), 1)
    BIGK = jnp.int32(1 << 30)
    HS = 4
    HH = SB // HS
    keys = [jnp.where(W != 0, (chunk << 16) | W, BIGK)[h * HH : (h + 1) * HH]
            for h in range(HS)]
    for k in range(K):
        for h in range(HS):
            key2 = keys[h]
            m2 = jnp.min(key2, axis=1, keepdims=True)
            w_sel = m2 & 0xFFFF
            mch = lax.shift_right_logical(m2, 16)
            low = w_sel & -w_sel
            pos = lax.shift_right_logical(
                lax.bitcast_convert_type(low.astype(jnp.float32), jnp.int32),
                23) - 127
            idx_k = mch * 16 + pos
            idx_ref[0, h * HH : (h + 1) * HH, k : k + 1] = jnp.where(
                m2 < BIGK, idx_k, -1)
            newkey = jnp.where(w_sel == low, BIGK, m2 ^ low)
            keys[h] = jnp.where(key2 == m2, newkey, key2)


def _ball_query_run(cx, cy, cz, xs, ys, zs, packmat):
    spec_c = pl.BlockSpec((1, SB, 1), lambda b, s: (b, s, 0))
    spec_x = pl.BlockSpec((1, 1, N), lambda b, s: (b, 0, 0))
    spec_p = pl.BlockSpec((N, NCH), lambda b, s: (0, 0))
    return pl.pallas_call(
        _bq_body,
        grid=(B, S // SB),
        in_specs=[spec_c, spec_c, spec_c, spec_x, spec_x, spec_x, spec_p],
        out_specs=pl.BlockSpec((1, SB, K), lambda b, s: (b, s, 0)),
        out_shape=jax.ShapeDtypeStruct((B, S, K), jnp.int32),
    )(cx, cy, cz, xs[:, None, :], ys[:, None, :], zs[:, None, :], packmat)


def _sc_gather(table, flat_idx):
    info = plsc.get_sparse_core_info()
    NC, NS = info.num_cores, info.num_subcores
    NW = NC * NS
    per_w = RTOT // NW
    CH = 128
    OUT_CH = 1024
    n_outer = per_w // OUT_CH
    n_inner = OUT_CH // CH
    idx2d = flat_idx.reshape(RTOT // CH, CH)
    mesh = plsc.VectorSubcoreMesh(core_axis_name="c", subcore_axis_name="s")

    @functools.partial(
        pl.kernel,
        out_type=jax.ShapeDtypeStruct((RTOT, DPAD), jnp.float32),
        mesh=mesh,
        scratch_types=[
            pltpu.VMEM((2, n_inner, CH), jnp.int32),
            pltpu.VMEM((2, OUT_CH, DPAD), jnp.float32),
            pltpu.SemaphoreType.DMA,
            pltpu.SemaphoreType.DMA,
            pltpu.SemaphoreType.DMA,
            pltpu.SemaphoreType.DMA,
            pltpu.SemaphoreType.DMA,
        ],
        compiler_params=pltpu.CompilerParams(use_tc_tiling_on_sc=False),
    )
    def k(table_hbm, idx_hbm, out_hbm, idx_v, rows_v, gsem, is0, is1, os0, os1):
        wid = lax.axis_index("s") * NC + lax.axis_index("c")
        row0 = wid * (per_w // CH)
        isem = [is0, is1]
        osem = [os0, os1]

        idx_c = [None, None]
        out_c = [None, None]
        idx_c[0] = pltpu.async_copy(
            idx_hbm.at[pl.ds(row0, n_inner)], idx_v.at[0], isem[0])
        for g in range(n_outer):
            bu = g & 1
            if g + 1 < n_outer:
                idx_c[1 - bu] = pltpu.async_copy(
                    idx_hbm.at[pl.ds(row0 + (g + 1) * n_inner, n_inner)],
                    idx_v.at[1 - bu], isem[1 - bu])
            idx_c[bu].wait()
            if out_c[bu] is not None:
                out_c[bu].wait()
            gcs = []
            for j in range(n_inner):
                gcs.append(pltpu.async_copy(
                    table_hbm.at[idx_v.at[bu, j]],
                    rows_v.at[bu, pl.ds(j * CH, CH)],
                    gsem,
                ))
            for c in gcs:
                c.wait()
            out_c[bu] = pltpu.async_copy(
                rows_v.at[bu],
                out_hbm.at[pl.ds(wid * per_w + g * OUT_CH, OUT_CH)],
                osem[bu])
        out_c[0].wait()
        out_c[1].wait()

    return k(table, idx2d)


def _m1_body(g_ref, c_ref, w_ref, wx_ref, b_ref, a_ref, ps_ref, pq_ref):
    g = g_ref[...]
    m = g[:, 19:20]
    crep = jnp.broadcast_to(
        c_ref[...][:, None, :], (RB // K, K, 8)).reshape(RB, 8)
    a = (jnp.dot(g, w_ref[...], preferred_element_type=jnp.float32)
         - jnp.dot(crep * m, wx_ref[...], preferred_element_type=jnp.float32)
         + b_ref[...])
    a_ref[...] = a
    ps_ref[0] = jnp.sum(a, axis=0, keepdims=True)
    pq_ref[0] = jnp.sum(a * a, axis=0, keepdims=True)


def _mlp1(gathered, cpad, w1p, wx, b1r):
    return pl.pallas_call(
        _m1_body,
        grid=(G,),
        in_specs=[
            pl.BlockSpec((RB, DPAD), lambda i: (i, 0)),
            pl.BlockSpec((RB // K, 8), lambda i: (i, 0)),
            pl.BlockSpec((DPAD, 32), lambda i: (0, 0)),
            pl.BlockSpec((8, 32), lambda i: (0, 0)),
            pl.BlockSpec((1, 32), lambda i: (0, 0)),
        ],
        out_specs=[
            pl.BlockSpec((RB, 32), lambda i: (i, 0)),
            pl.BlockSpec((1, 1, 32), lambda i: (i, 0, 0)),
            pl.BlockSpec((1, 1, 32), lambda i: (i, 0, 0)),
        ],
        out_shape=[
            jax.ShapeDtypeStruct((RTOT, 32), jnp.float32),
            jax.ShapeDtypeStruct((G, 1, 32), jnp.float32),
            jax.ShapeDtypeStruct((G, 1, 32), jnp.float32),
        ],
    )(gathered, cpad, w1p, wx, b1r)


def _mid_body(a_ref, ps_ref, pq_ref, g_ref, be_ref, w_ref, b_ref,
              o_ref, ops_ref, opq_ref):
    cnt = np.float32(RTOT)
    mean = jnp.sum(ps_ref[...], axis=0) / cnt
    ex2 = jnp.sum(pq_ref[...], axis=0) / cnt
    var = ex2 - mean * mean
    rstd = lax.rsqrt(var + EPSF)
    scale = g_ref[...] * rstd
    shift = be_ref[...] - mean * scale
    h = jnp.maximum(a_ref[...] * scale + shift, 0.0)
    o = jnp.dot(h, w_ref[...], preferred_element_type=jnp.float32) + b_ref[...]
    o_ref[...] = o
    ops_ref[0] = jnp.sum(o, axis=0, keepdims=True)
    opq_ref[0] = jnp.sum(o * o, axis=0, keepdims=True)


def _mlp_mid(a, ps, pq, gr, ber, wp, br, cout):
    cin = a.shape[1]
    return pl.pallas_call(
        _mid_body,
        grid=(G,),
        in_specs=[
            pl.BlockSpec((RB, cin), lambda i: (i, 0)),
            pl.BlockSpec((G, 1, cin), lambda i: (0, 0, 0)),
            pl.BlockSpec((G, 1, cin), lambda i: (0, 0, 0)),
            pl.BlockSpec((1, cin), lambda i: (0, 0)),
            pl.BlockSpec((1, cin), lambda i: (0, 0)),
            pl.BlockSpec((cin, cout), lambda i: (0, 0)),
            pl.BlockSpec((1, cout), lambda i: (0, 0)),
        ],
        out_specs=[
            pl.BlockSpec((RB, cout), lambda i: (i, 0)),
            pl.BlockSpec((1, 1, cout), lambda i: (i, 0, 0)),
            pl.BlockSpec((1, 1, cout), lambda i: (i, 0, 0)),
        ],
        out_shape=[
            jax.ShapeDtypeStruct((RTOT, cout), jnp.float32),
            jax.ShapeDtypeStruct((G, 1, cout), jnp.float32),
            jax.ShapeDtypeStruct((G, 1, cout), jnp.float32),
        ],
    )(a, ps, pq, gr, ber, wp, br)


def _fin_body(a_ref, ps_ref, pq_ref, g_ref, be_ref, o_ref):
    cnt = np.float32(RTOT)
    mean = jnp.sum(ps_ref[...], axis=0) / cnt
    ex2 = jnp.sum(pq_ref[...], axis=0) / cnt
    var = ex2 - mean * mean
    rstd = lax.rsqrt(var + EPSF)
    scale = g_ref[...] * rstd
    shift = be_ref[...] - mean * scale
    h = jnp.maximum(a_ref[...] * scale + shift, 0.0)
    o_ref[...] = jnp.max(h.reshape(RB // K, K, 64), axis=1)


def _mlp_fin(a, ps, pq, gr, ber):
    return pl.pallas_call(
        _fin_body,
        grid=(G,),
        in_specs=[
            pl.BlockSpec((RB, 64), lambda i: (i, 0)),
            pl.BlockSpec((G, 1, 64), lambda i: (0, 0, 0)),
            pl.BlockSpec((G, 1, 64), lambda i: (0, 0, 0)),
            pl.BlockSpec((1, 64), lambda i: (0, 0)),
            pl.BlockSpec((1, 64), lambda i: (0, 0)),
        ],
        out_specs=pl.BlockSpec((RB // K, 64), lambda i: (i, 0)),
        out_shape=jax.ShapeDtypeStruct((B * S, 64), jnp.float32),
    )(a, ps, pq, gr, ber)


def kernel(xyzs, feats, W1, b1, g1, be1, W2, b2, g2, be2, W3, b3, g3, be3):
    xs = xyzs[:, :, 0]
    ys = xyzs[:, :, 1]
    zs = xyzs[:, :, 2]

    idxs = _fps(xs, ys, zs)
    centers = jnp.take_along_axis(xyzs, idxs[..., None], axis=1)

    pmat = (np.left_shift(1, np.arange(N) % 16)[:, None]
            * (np.arange(N)[:, None] // 16 == np.arange(NCH)[None, :]))
    packmat = jnp.asarray(pmat, dtype=jnp.bfloat16)
    idx = _ball_query_run(centers[..., 0:1], centers[..., 1:2],
                          centers[..., 2:3], xs, ys, zs, packmat)

    table = jnp.concatenate(
        [xyzs, feats, jnp.ones((B, N, 1), jnp.float32),
         jnp.zeros((B, N, DPAD - 20), jnp.float32)], axis=-1
    ).reshape(B * N, DPAD)
    table = jnp.pad(table, ((0, 8), (0, 0)))
    boff = (jnp.arange(B, dtype=jnp.int32) * N)[:, None, None]
    flat_idx = jnp.where(idx >= 0, idx + boff, B * N).reshape(RTOT)
    gathered = _sc_gather(table, flat_idx)

    cpad = jnp.pad(centers.reshape(B * S, 3), ((0, 0), (0, 5)))

    w1p = jnp.pad(W1.T, ((0, DPAD - 19), (0, 0)))
    wx = jnp.pad(W1.T[:3], ((0, 5), (0, 0)))
    a1, ps1, pq1 = _mlp1(gathered, cpad, w1p, wx, b1.reshape(1, 32))

    a2, ps2, pq2 = _mlp_mid(a1, ps1, pq1, g1.reshape(1, 32), be1.reshape(1, 32),
                            W2.T, b2.reshape(1, 32), 32)
    a3, ps3, pq3 = _mlp_mid(a2, ps2, pq2, g2.reshape(1, 32), be2.reshape(1, 32),
                            W3.T, b3.reshape(1, 64), 64)
    cf = _mlp_fin(a3, ps3, pq3, g3.reshape(1, 64), be3.reshape(1, 64))

    return centers, cf.reshape(B, S, 64)


# speedup vs baseline: 19.1176x; 1.1381x over previous
import functools

import jax
import jax.numpy as jnp
import numpy as np
from jax import lax
from jax.experimental import pallas as pl
from jax.experimental.pallas import tpu as pltpu
from jax.experimental.pallas import tpu_sc as plsc

B = 8
N = 4096
S = 1024
K = 32
RTOT = B * S * K
R2 = np.float32(0.15) * np.float32(0.15)
EPSF = np.float32(1e-5)
RB = 4096
G = RTOT // RB
SB = 512
DPAD = 32


def _fps_body(xs_ref, ys_ref, zs_ref, idx_ref, mind_ref):
    X = xs_ref[...]
    Y = ys_ref[...]
    Z = zs_ref[...]
    lane = lax.broadcasted_iota(jnp.int32, (B, N), 1)
    mind_ref[...] = jnp.full((B, N), jnp.inf, dtype=jnp.float32)

    def body(i, carry):
        lx, ly, lz, acc, mind = carry
        dx = X - lx
        dy = Y - ly
        dz = Z - lz
        d2 = dx * dx + dy * dy + dz * dz
        mind = jnp.minimum(mind, d2)
        m = jnp.max(mind, axis=1, keepdims=True)
        nxt = jnp.min(jnp.where(mind == m, lane, N), axis=1, keepdims=True)
        oh = lane == nxt
        lx = jnp.sum(jnp.where(oh, X, 0.0), axis=1, keepdims=True)
        ly = jnp.sum(jnp.where(oh, Y, 0.0), axis=1, keepdims=True)
        lz = jnp.sum(jnp.where(oh, Z, 0.0), axis=1, keepdims=True)
        si = lax.broadcasted_iota(jnp.int32, (B, S), 1)
        acc = jnp.where(si == i, nxt, acc)
        return lx, ly, lz, acc, mind

    lx0 = X[:, 0:1]
    ly0 = Y[:, 0:1]
    lz0 = Z[:, 0:1]
    acc0 = jnp.zeros((B, S), dtype=jnp.int32)
    mind0 = jnp.full((B, N), jnp.inf, dtype=jnp.float32)
    _, _, _, acc, _ = lax.fori_loop(1, S, body, (lx0, ly0, lz0, acc0, mind0))
    idx_ref[...] = acc


def _fps(xs, ys, zs):
    return pl.pallas_call(
        _fps_body,
        out_shape=jax.ShapeDtypeStruct((B, S), jnp.int32),
        scratch_shapes=[pltpu.VMEM((B, N), jnp.float32)],
    )(xs, ys, zs)


NCH = N // 16


def _bq_body(cx_ref, cy_ref, cz_ref, xs_ref, ys_ref, zs_ref, p_ref, idx_ref):
    cx = cx_ref[0]
    cy = cy_ref[0]
    cz = cz_ref[0]
    X = xs_ref[0]
    Y = ys_ref[0]
    Z = zs_ref[0]
    dx = cx - X
    dy = cy - Y
    dz = cz - Z
    d2 = dx * dx + dy * dy + dz * dz
    within = d2 < R2
    wf = jnp.dot(within.astype(jnp.bfloat16), p_ref[...],
                 preferred_element_type=jnp.float32)
    W = wf.astype(jnp.int32)
    chunk = lax.broadcasted_iota(jnp.int32, (SB, NCH), 1)
    BIGK = jnp.int32(1 << 30)
    HS = 4
    HH = SB // HS
    keys = [jnp.where(W != 0, (chunk << 16) | W, BIGK)[h * HH : (h + 1) * HH]
            for h in range(HS)]
    for k in range(K):
        for h in range(HS):
            key2 = keys[h]
            m2 = jnp.min(key2, axis=1, keepdims=True)
            w_sel = m2 & 0xFFFF
            mch = lax.shift_right_logical(m2, 16)
            low = w_sel & -w_sel
            pos = lax.shift_right_logical(
                lax.bitcast_convert_type(low.astype(jnp.float32), jnp.int32),
                23) - 127
            idx_k = mch * 16 + pos
            idx_ref[0, h * HH : (h + 1) * HH, k : k + 1] = jnp.where(
                m2 < BIGK, idx_k, -1)
            newkey = jnp.where(w_sel == low, BIGK, m2 ^ low)
            keys[h] = jnp.where(key2 == m2, newkey, key2)


def _ball_query_run(cx, cy, cz, xs, ys, zs, packmat):
    spec_c = pl.BlockSpec((1, SB, 1), lambda b, s: (b, s, 0))
    spec_x = pl.BlockSpec((1, 1, N), lambda b, s: (b, 0, 0))
    spec_p = pl.BlockSpec((N, NCH), lambda b, s: (0, 0))
    return pl.pallas_call(
        _bq_body,
        grid=(B, S // SB),
        in_specs=[spec_c, spec_c, spec_c, spec_x, spec_x, spec_x, spec_p],
        out_specs=pl.BlockSpec((1, SB, K), lambda b, s: (b, s, 0)),
        out_shape=jax.ShapeDtypeStruct((B, S, K), jnp.int32),
    )(cx, cy, cz, xs[:, None, :], ys[:, None, :], zs[:, None, :], packmat)


def _sc_gather(table, flat_idx):
    info = plsc.get_sparse_core_info()
    NC, NS = info.num_cores, info.num_subcores
    NW = NC * NS
    per_w = RTOT // NW
    CH = 128
    OUT_CH = 1024
    n_outer = per_w // OUT_CH
    n_inner = OUT_CH // CH
    idx2d = flat_idx.reshape(RTOT // CH, CH)
    mesh = plsc.VectorSubcoreMesh(core_axis_name="c", subcore_axis_name="s")

    @functools.partial(
        pl.kernel,
        out_type=jax.ShapeDtypeStruct((RTOT, DPAD), jnp.float32),
        mesh=mesh,
        scratch_types=[
            pltpu.VMEM((2, n_inner, CH), jnp.int32),
            pltpu.VMEM((2, OUT_CH, DPAD), jnp.float32),
            pltpu.SemaphoreType.DMA,
            pltpu.SemaphoreType.DMA,
            pltpu.SemaphoreType.DMA,
            pltpu.SemaphoreType.DMA,
            pltpu.SemaphoreType.DMA,
        ],
        compiler_params=pltpu.CompilerParams(use_tc_tiling_on_sc=False),
    )
    def k(table_hbm, idx_hbm, out_hbm, idx_v, rows_v, gsem, is0, is1, os0, os1):
        wid = lax.axis_index("s") * NC + lax.axis_index("c")
        row0 = wid * (per_w // CH)
        isem = [is0, is1]
        osem = [os0, os1]

        idx_c = [None, None]
        out_c = [None, None]
        idx_c[0] = pltpu.async_copy(
            idx_hbm.at[pl.ds(row0, n_inner)], idx_v.at[0], isem[0])
        for g in range(n_outer):
            bu = g & 1
            if g + 1 < n_outer:
                idx_c[1 - bu] = pltpu.async_copy(
                    idx_hbm.at[pl.ds(row0 + (g + 1) * n_inner, n_inner)],
                    idx_v.at[1 - bu], isem[1 - bu])
            idx_c[bu].wait()
            if out_c[bu] is not None:
                out_c[bu].wait()
            gcs = []
            for j in range(n_inner):
                gcs.append(pltpu.async_copy(
                    table_hbm.at[idx_v.at[bu, j]],
                    rows_v.at[bu, pl.ds(j * CH, CH)],
                    gsem,
                ))
            for c in gcs:
                c.wait()
            out_c[bu] = pltpu.async_copy(
                rows_v.at[bu],
                out_hbm.at[pl.ds(wid * per_w + g * OUT_CH, OUT_CH)],
                osem[bu])
        out_c[0].wait()
        out_c[1].wait()

    return k(table, idx2d)


def _m1_body(g_ref, c_ref, w_ref, wx_ref, b_ref, a_ref, ps_ref, pq_ref):
    g = g_ref[...]
    m = g[:, 19:20]
    crep = jnp.broadcast_to(
        c_ref[...][:, None, :], (RB // K, K, 8)).reshape(RB, 8)
    a = (jnp.dot(g, w_ref[...], preferred_element_type=jnp.float32)
         - jnp.dot(crep * m, wx_ref[...], preferred_element_type=jnp.float32)
         + b_ref[...])
    a_ref[...] = a
    ps_ref[0] = jnp.sum(a, axis=0, keepdims=True)
    pq_ref[0] = jnp.sum(a * a, axis=0, keepdims=True)


def _mlp1(gathered, cpad, w1p, wx, b1r):
    return pl.pallas_call(
        _m1_body,
        grid=(G,),
        in_specs=[
            pl.BlockSpec((RB, DPAD), lambda i: (i, 0)),
            pl.BlockSpec((RB // K, 8), lambda i: (i, 0)),
            pl.BlockSpec((DPAD, 32), lambda i: (0, 0)),
            pl.BlockSpec((8, 32), lambda i: (0, 0)),
            pl.BlockSpec((1, 32), lambda i: (0, 0)),
        ],
        out_specs=[
            pl.BlockSpec((RB, 32), lambda i: (i, 0)),
            pl.BlockSpec((1, 1, 32), lambda i: (i, 0, 0)),
            pl.BlockSpec((1, 1, 32), lambda i: (i, 0, 0)),
        ],
        out_shape=[
            jax.ShapeDtypeStruct((RTOT, 32), jnp.float32),
            jax.ShapeDtypeStruct((G, 1, 32), jnp.float32),
            jax.ShapeDtypeStruct((G, 1, 32), jnp.float32),
        ],
    )(gathered, cpad, w1p, wx, b1r)


def _mid_body(a_ref, ps_ref, pq_ref, g_ref, be_ref, w_ref, b_ref,
              o_ref, ops_ref, opq_ref):
    cnt = np.float32(RTOT)
    mean = jnp.sum(ps_ref[...], axis=0) / cnt
    ex2 = jnp.sum(pq_ref[...], axis=0) / cnt
    var = ex2 - mean * mean
    rstd = lax.rsqrt(var + EPSF)
    scale = g_ref[...] * rstd
    shift = be_ref[...] - mean * scale
    h = jnp.maximum(a_ref[...] * scale + shift, 0.0)
    o = jnp.dot(h, w_ref[...], preferred_element_type=jnp.float32) + b_ref[...]
    o_ref[...] = o
    ops_ref[0] = jnp.sum(o, axis=0, keepdims=True)
    opq_ref[0] = jnp.sum(o * o, axis=0, keepdims=True)


def _mlp_mid(a, ps, pq, gr, ber, wp, br, cout):
    cin = a.shape[1]
    return pl.pallas_call(
        _mid_body,
        grid=(G,),
        in_specs=[
            pl.BlockSpec((RB, cin), lambda i: (i, 0)),
            pl.BlockSpec((G, 1, cin), lambda i: (0, 0, 0)),
            pl.BlockSpec((G, 1, cin), lambda i: (0, 0, 0)),
            pl.BlockSpec((1, cin), lambda i: (0, 0)),
            pl.BlockSpec((1, cin), lambda i: (0, 0)),
            pl.BlockSpec((cin, cout), lambda i: (0, 0)),
            pl.BlockSpec((1, cout), lambda i: (0, 0)),
        ],
        out_specs=[
            pl.BlockSpec((RB, cout), lambda i: (i, 0)),
            pl.BlockSpec((1, 1, cout), lambda i: (i, 0, 0)),
            pl.BlockSpec((1, 1, cout), lambda i: (i, 0, 0)),
        ],
        out_shape=[
            jax.ShapeDtypeStruct((RTOT, cout), jnp.float32),
            jax.ShapeDtypeStruct((G, 1, cout), jnp.float32),
            jax.ShapeDtypeStruct((G, 1, cout), jnp.float32),
        ],
    )(a, ps, pq, gr, ber, wp, br)


def _fin_body(a_ref, ps_ref, pq_ref, g_ref, be_ref, o_ref):
    cnt = np.float32(RTOT)
    mean = jnp.sum(ps_ref[...], axis=0) / cnt
    ex2 = jnp.sum(pq_ref[...], axis=0) / cnt
    var = ex2 - mean * mean
    rstd = lax.rsqrt(var + EPSF)
    scale = g_ref[...] * rstd
    shift = be_ref[...] - mean * scale
    h = jnp.maximum(a_ref[...] * scale + shift, 0.0)
    o_ref[...] = jnp.max(h.reshape(RB // K, K, 64), axis=1)


def _mlp_fin(a, ps, pq, gr, ber):
    return pl.pallas_call(
        _fin_body,
        grid=(G,),
        in_specs=[
            pl.BlockSpec((RB, 64), lambda i: (i, 0)),
            pl.BlockSpec((G, 1, 64), lambda i: (0, 0, 0)),
            pl.BlockSpec((G, 1, 64), lambda i: (0, 0, 0)),
            pl.BlockSpec((1, 64), lambda i: (0, 0)),
            pl.BlockSpec((1, 64), lambda i: (0, 0)),
        ],
        out_specs=pl.BlockSpec((RB // K, 64), lambda i: (i, 0)),
        out_shape=jax.ShapeDtypeStruct((B * S, 64), jnp.float32),
    )(a, ps, pq, gr, ber)


def kernel(xyzs, feats, W1, b1, g1, be1, W2, b2, g2, be2, W3, b3, g3, be3):
    xs = xyzs[:, :, 0]
    ys = xyzs[:, :, 1]
    zs = xyzs[:, :, 2]

    idxs = _fps(xs, ys, zs)
    centers = jnp.take_along_axis(xyzs, idxs[..., None], axis=1)

    pmat = (np.left_shift(1, np.arange(N) % 16)[:, None]
            * (np.arange(N)[:, None] // 16 == np.arange(NCH)[None, :]))
    packmat = jnp.asarray(pmat, dtype=jnp.bfloat16)
    idx = _ball_query_run(centers[..., 0:1], centers[..., 1:2],
                          centers[..., 2:3], xs, ys, zs, packmat)

    table = jnp.concatenate(
        [xyzs, feats, jnp.ones((B, N, 1), jnp.float32),
         jnp.zeros((B, N, DPAD - 20), jnp.float32)], axis=-1
    ).reshape(B * N, DPAD)
    table = jnp.pad(table, ((0, 8), (0, 0)))
    boff = (jnp.arange(B, dtype=jnp.int32) * N)[:, None, None]
    flat_idx = jnp.where(idx >= 0, idx + boff, B * N).reshape(RTOT)
    gathered = _sc_gather(table, flat_idx)

    cpad = jnp.pad(centers.reshape(B * S, 3), ((0, 0), (0, 5)))

    w1p = jnp.pad(W1.T, ((0, DPAD - 19), (0, 0)))
    wx = jnp.pad(W1.T[:3], ((0, 5), (0, 0)))
    a1, ps1, pq1 = _mlp1(gathered, cpad, w1p, wx, b1.reshape(1, 32))

    a2, ps2, pq2 = _mlp_mid(a1, ps1, pq1, g1.reshape(1, 32), be1.reshape(1, 32),
                            W2.T, b2.reshape(1, 32), 32)
    a3, ps3, pq3 = _mlp_mid(a2, ps2, pq2, g2.reshape(1, 32), be2.reshape(1, 32),
                            W3.T, b3.reshape(1, 64), 64)
    cf = _mlp_fin(a3, ps3, pq3, g3.reshape(1, 64), be3.reshape(1, 64))

    return centers, cf.reshape(B, S, 64)


# speedup vs baseline: 20.1596x; 1.0545x over previous
import functools

import jax
import jax.numpy as jnp
import numpy as np
from jax import lax
from jax.experimental import pallas as pl
from jax.experimental.pallas import tpu as pltpu
from jax.experimental.pallas import tpu_sc as plsc

B = 8
N = 4096
S = 1024
K = 32
RTOT = B * S * K
R2 = np.float32(0.15) * np.float32(0.15)
EPSF = np.float32(1e-5)
RB = 8192
G = RTOT // RB
SB = 512
DPAD = 32


def _fps_body(xs_ref, ys_ref, zs_ref, idx_ref, mind_ref):
    X = xs_ref[...]
    Y = ys_ref[...]
    Z = zs_ref[...]
    lane = lax.broadcasted_iota(jnp.int32, (B, N), 1)
    mind_ref[...] = jnp.full((B, N), jnp.inf, dtype=jnp.float32)

    def body(i, carry):
        lx, ly, lz, acc, mind = carry
        dx = X - lx
        dy = Y - ly
        dz = Z - lz
        d2 = dx * dx + dy * dy + dz * dz
        mind = jnp.minimum(mind, d2)
        m = jnp.max(mind, axis=1, keepdims=True)
        nxt = jnp.min(jnp.where(mind == m, lane, N), axis=1, keepdims=True)
        oh = lane == nxt
        lx = jnp.sum(jnp.where(oh, X, 0.0), axis=1, keepdims=True)
        ly = jnp.sum(jnp.where(oh, Y, 0.0), axis=1, keepdims=True)
        lz = jnp.sum(jnp.where(oh, Z, 0.0), axis=1, keepdims=True)
        si = lax.broadcasted_iota(jnp.int32, (B, S), 1)
        acc = jnp.where(si == i, nxt, acc)
        return lx, ly, lz, acc, mind

    lx0 = X[:, 0:1]
    ly0 = Y[:, 0:1]
    lz0 = Z[:, 0:1]
    acc0 = jnp.zeros((B, S), dtype=jnp.int32)
    mind0 = jnp.full((B, N), jnp.inf, dtype=jnp.float32)
    _, _, _, acc, _ = lax.fori_loop(1, S, body, (lx0, ly0, lz0, acc0, mind0))
    idx_ref[...] = acc


def _fps(xs, ys, zs):
    return pl.pallas_call(
        _fps_body,
        out_shape=jax.ShapeDtypeStruct((B, S), jnp.int32),
        scratch_shapes=[pltpu.VMEM((B, N), jnp.float32)],
    )(xs, ys, zs)


NCH = N // 16


def _bq_body(cx_ref, cy_ref, cz_ref, xs_ref, ys_ref, zs_ref, p_ref, idx_ref):
    cx = cx_ref[0]
    cy = cy_ref[0]
    cz = cz_ref[0]
    X = xs_ref[0]
    Y = ys_ref[0]
    Z = zs_ref[0]
    dx = cx - X
    dy = cy - Y
    dz = cz - Z
    d2 = dx * dx + dy * dy + dz * dz
    within = d2 < R2
    wf = jnp.dot(within.astype(jnp.bfloat16), p_ref[...],
                 preferred_element_type=jnp.float32)
    W = wf.astype(jnp.int32)
    chunk = lax.broadcasted_iota(jnp.int32, (SB, NCH), 1)
    BIGK = jnp.int32(1 << 30)
    HS = 4
    HH = SB // HS
    keys = [jnp.where(W != 0, (chunk << 16) | W, BIGK)[h * HH : (h + 1) * HH]
            for h in range(HS)]
    for k in range(K):
        for h in range(HS):
            key2 = keys[h]
            m2 = jnp.min(key2, axis=1, keepdims=True)
            w_sel = m2 & 0xFFFF
            mch = lax.shift_right_logical(m2, 16)
            low = w_sel & -w_sel
            pos = lax.shift_right_logical(
                lax.bitcast_convert_type(low.astype(jnp.float32), jnp.int32),
                23) - 127
            idx_k = mch * 16 + pos
            idx_ref[0, h * HH : (h + 1) * HH, k : k + 1] = jnp.where(
                m2 < BIGK, idx_k, -1)
            newkey = jnp.where(w_sel == low, BIGK, m2 ^ low)
            keys[h] = jnp.where(key2 == m2, newkey, key2)


def _ball_query_run(cx, cy, cz, xs, ys, zs, packmat):
    spec_c = pl.BlockSpec((1, SB, 1), lambda b, s: (b, s, 0))
    spec_x = pl.BlockSpec((1, 1, N), lambda b, s: (b, 0, 0))
    spec_p = pl.BlockSpec((N, NCH), lambda b, s: (0, 0))
    return pl.pallas_call(
        _bq_body,
        grid=(B, S // SB),
        in_specs=[spec_c, spec_c, spec_c, spec_x, spec_x, spec_x, spec_p],
        out_specs=pl.BlockSpec((1, SB, K), lambda b, s: (b, s, 0)),
        out_shape=jax.ShapeDtypeStruct((B, S, K), jnp.int32),
    )(cx, cy, cz, xs[:, None, :], ys[:, None, :], zs[:, None, :], packmat)


def _sc_gather(table, flat_idx):
    info = plsc.get_sparse_core_info()
    NC, NS = info.num_cores, info.num_subcores
    NW = NC * NS
    per_w = RTOT // NW
    CH = 128
    OUT_CH = 1024
    n_outer = per_w // OUT_CH
    n_inner = OUT_CH // CH
    idx2d = flat_idx.reshape(RTOT // CH, CH)
    mesh = plsc.VectorSubcoreMesh(core_axis_name="c", subcore_axis_name="s")

    @functools.partial(
        pl.kernel,
        out_type=jax.ShapeDtypeStruct((RTOT, DPAD), jnp.float32),
        mesh=mesh,
        scratch_types=[
            pltpu.VMEM((2, n_inner, CH), jnp.int32),
            pltpu.VMEM((2, OUT_CH, DPAD), jnp.float32),
            pltpu.SemaphoreType.DMA,
            pltpu.SemaphoreType.DMA,
            pltpu.SemaphoreType.DMA,
            pltpu.SemaphoreType.DMA,
            pltpu.SemaphoreType.DMA,
        ],
        compiler_params=pltpu.CompilerParams(use_tc_tiling_on_sc=False),
    )
    def k(table_hbm, idx_hbm, out_hbm, idx_v, rows_v, gsem, is0, is1, os0, os1):
        wid = lax.axis_index("s") * NC + lax.axis_index("c")
        row0 = wid * (per_w // CH)
        isem = [is0, is1]
        osem = [os0, os1]

        idx_c = [None, None]
        out_c = [None, None]
        idx_c[0] = pltpu.async_copy(
            idx_hbm.at[pl.ds(row0, n_inner)], idx_v.at[0], isem[0])
        for g in range(n_outer):
            bu = g & 1
            if g + 1 < n_outer:
                idx_c[1 - bu] = pltpu.async_copy(
                    idx_hbm.at[pl.ds(row0 + (g + 1) * n_inner, n_inner)],
                    idx_v.at[1 - bu], isem[1 - bu])
            idx_c[bu].wait()
            if out_c[bu] is not None:
                out_c[bu].wait()
            gcs = []
            for j in range(n_inner):
                gcs.append(pltpu.async_copy(
                    table_hbm.at[idx_v.at[bu, j]],
                    rows_v.at[bu, pl.ds(j * CH, CH)],
                    gsem,
                ))
            for c in gcs:
                c.wait()
            out_c[bu] = pltpu.async_copy(
                rows_v.at[bu],
                out_hbm.at[pl.ds(wid * per_w + g * OUT_CH, OUT_CH)],
                osem[bu])
        out_c[0].wait()
        out_c[1].wait()

    return k(table, idx2d)


def _m1_body(g_ref, c_ref, w_ref, wx_ref, b_ref, a_ref, ps_ref, pq_ref):
    g = g_ref[...]
    m = g[:, 19:20]
    crep = jnp.broadcast_to(
        c_ref[...][:, None, :], (RB // K, K, 8)).reshape(RB, 8)
    a = (jnp.dot(g, w_ref[...], preferred_element_type=jnp.float32)
         - jnp.dot(crep * m, wx_ref[...], preferred_element_type=jnp.float32)
         + b_ref[...])
    a_ref[...] = a
    ps_ref[0] = jnp.sum(a, axis=0, keepdims=True)
    pq_ref[0] = jnp.sum(a * a, axis=0, keepdims=True)


def _mlp1(gathered, cpad, w1p, wx, b1r):
    return pl.pallas_call(
        _m1_body,
        grid=(G,),
        in_specs=[
            pl.BlockSpec((RB, DPAD), lambda i: (i, 0)),
            pl.BlockSpec((RB // K, 8), lambda i: (i, 0)),
            pl.BlockSpec((DPAD, 32), lambda i: (0, 0)),
            pl.BlockSpec((8, 32), lambda i: (0, 0)),
            pl.BlockSpec((1, 32), lambda i: (0, 0)),
        ],
        out_specs=[
            pl.BlockSpec((RB, 32), lambda i: (i, 0)),
            pl.BlockSpec((1, 1, 32), lambda i: (i, 0, 0)),
            pl.BlockSpec((1, 1, 32), lambda i: (i, 0, 0)),
        ],
        out_shape=[
            jax.ShapeDtypeStruct((RTOT, 32), jnp.float32),
            jax.ShapeDtypeStruct((G, 1, 32), jnp.float32),
            jax.ShapeDtypeStruct((G, 1, 32), jnp.float32),
        ],
    )(gathered, cpad, w1p, wx, b1r)


def _mid_body(a_ref, ps_ref, pq_ref, g_ref, be_ref, w_ref, b_ref,
              o_ref, ops_ref, opq_ref):
    cnt = np.float32(RTOT)
    mean = jnp.sum(ps_ref[...], axis=0) / cnt
    ex2 = jnp.sum(pq_ref[...], axis=0) / cnt
    var = ex2 - mean * mean
    rstd = lax.rsqrt(var + EPSF)
    scale = g_ref[...] * rstd
    shift = be_ref[...] - mean * scale
    h = jnp.maximum(a_ref[...] * scale + shift, 0.0)
    o = jnp.dot(h, w_ref[...], preferred_element_type=jnp.float32) + b_ref[...]
    o_ref[...] = o
    ops_ref[0] = jnp.sum(o, axis=0, keepdims=True)
    opq_ref[0] = jnp.sum(o * o, axis=0, keepdims=True)


def _mlp_mid(a, ps, pq, gr, ber, wp, br, cout):
    cin = a.shape[1]
    return pl.pallas_call(
        _mid_body,
        grid=(G,),
        in_specs=[
            pl.BlockSpec((RB, cin), lambda i: (i, 0)),
            pl.BlockSpec((G, 1, cin), lambda i: (0, 0, 0)),
            pl.BlockSpec((G, 1, cin), lambda i: (0, 0, 0)),
            pl.BlockSpec((1, cin), lambda i: (0, 0)),
            pl.BlockSpec((1, cin), lambda i: (0, 0)),
            pl.BlockSpec((cin, cout), lambda i: (0, 0)),
            pl.BlockSpec((1, cout), lambda i: (0, 0)),
        ],
        out_specs=[
            pl.BlockSpec((RB, cout), lambda i: (i, 0)),
            pl.BlockSpec((1, 1, cout), lambda i: (i, 0, 0)),
            pl.BlockSpec((1, 1, cout), lambda i: (i, 0, 0)),
        ],
        out_shape=[
            jax.ShapeDtypeStruct((RTOT, cout), jnp.float32),
            jax.ShapeDtypeStruct((G, 1, cout), jnp.float32),
            jax.ShapeDtypeStruct((G, 1, cout), jnp.float32),
        ],
    )(a, ps, pq, gr, ber, wp, br)


def _fin_body(a_ref, ps_ref, pq_ref, g_ref, be_ref, o_ref):
    cnt = np.float32(RTOT)
    mean = jnp.sum(ps_ref[...], axis=0) / cnt
    ex2 = jnp.sum(pq_ref[...], axis=0) / cnt
    var = ex2 - mean * mean
    rstd = lax.rsqrt(var + EPSF)
    scale = g_ref[...] * rstd
    shift = be_ref[...] - mean * scale
    h = jnp.maximum(a_ref[...] * scale + shift, 0.0)
    o_ref[...] = jnp.max(h.reshape(RB // K, K, 64), axis=1)


def _mlp_fin(a, ps, pq, gr, ber):
    return pl.pallas_call(
        _fin_body,
        grid=(G,),
        in_specs=[
            pl.BlockSpec((RB, 64), lambda i: (i, 0)),
            pl.BlockSpec((G, 1, 64), lambda i: (0, 0, 0)),
            pl.BlockSpec((G, 1, 64), lambda i: (0, 0, 0)),
            pl.BlockSpec((1, 64), lambda i: (0, 0)),
            pl.BlockSpec((1, 64), lambda i: (0, 0)),
        ],
        out_specs=pl.BlockSpec((RB // K, 64), lambda i: (i, 0)),
        out_shape=jax.ShapeDtypeStruct((B * S, 64), jnp.float32),
    )(a, ps, pq, gr, ber)


def kernel(xyzs, feats, W1, b1, g1, be1, W2, b2, g2, be2, W3, b3, g3, be3):
    xs = xyzs[:, :, 0]
    ys = xyzs[:, :, 1]
    zs = xyzs[:, :, 2]

    idxs = _fps(xs, ys, zs)
    centers = jnp.take_along_axis(xyzs, idxs[..., None], axis=1)

    pmat = (np.left_shift(1, np.arange(N) % 16)[:, None]
            * (np.arange(N)[:, None] // 16 == np.arange(NCH)[None, :]))
    packmat = jnp.asarray(pmat, dtype=jnp.bfloat16)
    idx = _ball_query_run(centers[..., 0:1], centers[..., 1:2],
                          centers[..., 2:3], xs, ys, zs, packmat)

    table = jnp.concatenate(
        [xyzs, feats, jnp.ones((B, N, 1), jnp.float32),
         jnp.zeros((B, N, DPAD - 20), jnp.float32)], axis=-1
    ).reshape(B * N, DPAD)
    table = jnp.pad(table, ((0, 8), (0, 0)))
    boff = (jnp.arange(B, dtype=jnp.int32) * N)[:, None, None]
    flat_idx = jnp.where(idx >= 0, idx + boff, B * N).reshape(RTOT)
    gathered = _sc_gather(table, flat_idx)

    cpad = jnp.pad(centers.reshape(B * S, 3), ((0, 0), (0, 5)))

    w1p = jnp.pad(W1.T, ((0, DPAD - 19), (0, 0)))
    wx = jnp.pad(W1.T[:3], ((0, 5), (0, 0)))
    a1, ps1, pq1 = _mlp1(gathered, cpad, w1p, wx, b1.reshape(1, 32))

    a2, ps2, pq2 = _mlp_mid(a1, ps1, pq1, g1.reshape(1, 32), be1.reshape(1, 32),
                            W2.T, b2.reshape(1, 32), 32)
    a3, ps3, pq3 = _mlp_mid(a2, ps2, pq2, g2.reshape(1, 32), be2.reshape(1, 32),
                            W3.T, b3.reshape(1, 64), 64)
    cf = _mlp_fin(a3, ps3, pq3, g3.reshape(1, 64), be3.reshape(1, 64))

    return centers, cf.reshape(B, S, 64)
